# Initial kernel scaffold; baseline (speedup 1.0000x reference)
import jax
import jax.numpy as jnp
from jax.experimental import pallas as pl


def kernel(x, edge_index, edge_weight, Wz, bz, Wr, br, Wh, bh, LzW, Lzb, LrW, Lrb, LhW, Lhb, att, linW, linb):
    raise NotImplementedError("write your pallas kernel here")


# speedup vs baseline: 9.0667x; 9.0667x over previous
import functools
import jax
import jax.numpy as jnp
from jax import lax
from jax.experimental import pallas as pl
from jax.experimental.pallas import tpu as pltpu
from jax.experimental.pallas import tpu_sc as plsc

N = 100000
E = 1600000
F_IN = 8
F_OUT = 24
P = 12
NP = 102400
EP = 1638400
ER = 12800
NCH = 10
CH = NP // NCH
CHT = CH + 128
TRASH = CH
YTR = CHT // 16
ZR = 8
DR = CH // 16
SROWS = ER // 16
SC_SUP = 50
STG = 2208

_mesh = plsc.VectorSubcoreMesh(core_axis_name="c", subcore_axis_name="s")


def _splat(vec, lane):
    idx = jnp.full((16, 1), lane, jnp.int32)
    return lax.gather(
        vec, idx,
        lax.GatherDimensionNumbers(offset_dims=(), collapsed_slice_dims=(0,),
                                   start_index_map=(0,)),
        (1,), mode=lax.GatherScatterMode.PROMISE_IN_BOUNDS)


@functools.partial(
    pl.kernel, mesh=_mesh,
    compiler_params=pltpu.CompilerParams(needs_layout_passes=False),
    out_type=jax.ShapeDtypeStruct((NP, 128), jnp.float32),
    scratch_types=[
        pltpu.VMEM_SHARED((CHT, 128), jnp.float32),
        pltpu.VMEM((16, 128), jnp.int32),
        pltpu.VMEM((16, 128), jnp.float32),
        pltpu.VMEM((STG,), jnp.int32),
        pltpu.VMEM((STG,), jnp.float32),
        pltpu.VMEM((128,), jnp.int32),
        pltpu.VMEM((128,), jnp.int32),
        pltpu.VMEM((128, 128), jnp.float32),
        pltpu.VMEM((128, 128), jnp.float32),
        pltpu.VMEM((ZR, 128), jnp.float32),
        pltpu.SemaphoreType.DMA,
        pltpu.SemaphoreType.DMA,
    ],
)
def _deg_kernel(col_h, w_h, deg_out, ychunk, cbuf, wbuf, s_l, s_w,
                lrow0, lrow1, rows_v0, rows_v1, zbuf, sems0, sems1):
    core = lax.axis_index("c")
    tid = lax.axis_index("s")

    def zb(j, carry):
        zbuf[j // 8, pl.ds((j % 8) * 16, 16)] = jnp.zeros((16,), jnp.float32)
        return carry
    lax.fori_loop(0, ZR * 8, zb, 0)

    def zr(j, carry):
        rows_v0[j // 8, pl.ds((j % 8) * 16, 16)] = jnp.zeros((16,),
                                                             jnp.float32)
        rows_v1[j // 8, pl.ds((j % 8) * 16, 16)] = jnp.zeros((16,),
                                                             jnp.float32)
        return carry
    lax.fori_loop(0, 128 * 8, zr, 0)

    def one_pass(p, carry):
        fc = core * (NCH // 2) + p
        lo = fc * CH

        def zy(m, c2):
            pltpu.sync_copy(zbuf, ychunk.at[pl.ds(tid * YTR + m * ZR, ZR), :])
            return c2
        lax.fori_loop(0, YTR // ZR, zy, 0)
        plsc.subcore_barrier()

        def schunk(k, c2):
            base = tid * SROWS + k * 16
            pltpu.sync_copy(col_h.at[pl.ds(base, 16)], cbuf)
            pltpu.sync_copy(w_h.at[pl.ds(base, 16)], wbuf)
            for r in range(16):
                for l in range(8):
                    e16 = (r * 8 + l) * 16
                    colv = cbuf[r, pl.ds(l * 16, 16)]
                    wv = wbuf[r, pl.ds(l * 16, 16)]
                    m = (colv >= lo) & (colv < lo + CH)
                    s_l[pl.ds(e16, 16)] = jnp.where(m, colv - lo, TRASH)
                    s_w[pl.ds(e16, 16)] = jnp.where(m, wv, 0.0)

            lrows = (lrow0, lrow1)
            rbufs = (rows_v0, rows_v1)
            semss = (sems0, sems1)

            def pair(k2, c3):
                sh = []
                for pb in range(2):
                    b = k2 * 2 + pb
                    for i in range(8):
                        lrows[pb][pl.ds(i * 16, 16)] = (
                            s_l[pl.ds(b * 128 + i * 16, 16)])
                    for r in range(8):
                        wv = s_w[pl.ds(b * 128 + r * 16, 16)]
                        for l2 in range(16):
                            rbufs[pb][r * 16 + l2, pl.ds(0, 16)] = (
                                _splat(wv, l2))
                    sh.append(pltpu.async_copy(rbufs[pb],
                                               ychunk.at[lrows[pb]],
                                               semss[pb], add=True))
                sh[0].wait()
                sh[1].wait()
                return c3
            lax.fori_loop(0, 8, pair, 0)
            return c2
        lax.fori_loop(0, SC_SUP, schunk, 0)
        plsc.subcore_barrier()
        pltpu.sync_copy(ychunk.at[pl.ds(tid * DR, DR), :],
                        deg_out.at[pl.ds(lo + tid * DR, DR), :])
        plsc.subcore_barrier()
        return carry
    lax.fori_loop(0, NCH // 2, one_pass, 0)


@functools.partial(
    pl.kernel, mesh=_mesh,
    compiler_params=pltpu.CompilerParams(needs_layout_passes=False),
    out_type=jax.ShapeDtypeStruct((NP, 128), jnp.float32),
    scratch_types=[
        pltpu.VMEM_SHARED((CHT, 128), jnp.float32),
        pltpu.VMEM((16, 128), jnp.int32),
        pltpu.VMEM((16, 128), jnp.int32),
        pltpu.VMEM((16, 128), jnp.float32),
        pltpu.VMEM((STG,), jnp.int32),
        pltpu.VMEM((STG,), jnp.int32),
        pltpu.VMEM((STG,), jnp.float32),
        pltpu.VMEM((128,), jnp.int32),
        pltpu.VMEM((128,), jnp.int32),
        pltpu.VMEM((128,), jnp.int32),
        pltpu.VMEM((128,), jnp.int32),
        pltpu.VMEM((128, 128), jnp.float32),
        pltpu.VMEM((128, 128), jnp.float32),
        pltpu.VMEM((ZR, 128), jnp.float32),
        pltpu.SemaphoreType.DMA,
        pltpu.SemaphoreType.DMA,
        pltpu.SemaphoreType.DMA,
        pltpu.SemaphoreType.DMA,
    ],
)
def _scatter_kernel(row_h, col_h, w_h, xs_h, s_out, ychunk,
                    rbuf, cbuf, wbuf, s_r, s_l, s_w, grow0, grow1,
                    lrow0, lrow1, rows_v0, rows_v1, zbuf,
                    semg0, semg1, sems0, sems1):
    core = lax.axis_index("c")
    tid = lax.axis_index("s")

    def zb(j, carry):
        zbuf[j // 8, pl.ds((j % 8) * 16, 16)] = jnp.zeros((16,), jnp.float32)
        return carry
    lax.fori_loop(0, ZR * 8, zb, 0)

    def one_pass(p, carry):
        fc = core * (NCH // 2) + p
        lo = fc * CH

        def zy(m, c2):
            pltpu.sync_copy(zbuf, ychunk.at[pl.ds(tid * YTR + m * ZR, ZR), :])
            return c2
        lax.fori_loop(0, YTR // ZR, zy, 0)
        plsc.subcore_barrier()

        def schunk(k, c2):
            base = tid * SROWS + k * 16
            pltpu.sync_copy(row_h.at[pl.ds(base, 16)], rbuf)
            pltpu.sync_copy(col_h.at[pl.ds(base, 16)], cbuf)
            pltpu.sync_copy(w_h.at[pl.ds(base, 16)], wbuf)
            for r in range(16):
                for l in range(8):
                    e16 = (r * 8 + l) * 16
                    colv = cbuf[r, pl.ds(l * 16, 16)]
                    rowv = rbuf[r, pl.ds(l * 16, 16)]
                    wv = wbuf[r, pl.ds(l * 16, 16)]
                    m = (colv >= lo) & (colv < lo + CH)
                    s_l[pl.ds(e16, 16)] = jnp.where(m, colv - lo, TRASH)
                    s_r[pl.ds(e16, 16)] = rowv
                    s_w[pl.ds(e16, 16)] = jnp.where(m, wv, 0.0)

            grows = (grow0, grow1)
            lrows = (lrow0, lrow1)
            rbufs = (rows_v0, rows_v1)
            semgs = (semg0, semg1)
            semss = (sems0, sems1)

            def pair(k2, c3):
                for pb in range(2):
                    b = k2 * 2 + pb
                    for i in range(8):
                        grows[pb][pl.ds(i * 16, 16)] = (
                            s_r[pl.ds(b * 128 + i * 16, 16)])
                        lrows[pb][pl.ds(i * 16, 16)] = (
                            s_l[pl.ds(b * 128 + i * 16, 16)])
                gh = [pltpu.async_copy(xs_h.at[grows[pb]], rbufs[pb],
                                       semgs[pb]) for pb in range(2)]
                sh = []
                for pb in range(2):
                    b = k2 * 2 + pb
                    cur = rbufs[pb]
                    gh[pb].wait()
                    for r in range(8):
                        wv = s_w[pl.ds(b * 128 + r * 16, 16)]
                        for l2 in range(16):
                            e = r * 16 + l2
                            s = _splat(wv, l2)
                            for q in range(6):
                                cur[e, pl.ds(q * 16, 16)] = (
                                    cur[e, pl.ds(q * 16, 16)] * s)
                    sh.append(pltpu.async_copy(cur, ychunk.at[lrows[pb]],
                                               semss[pb], add=True))
                sh[0].wait()
                sh[1].wait()
                return c3
            lax.fori_loop(0, 8, pair, 0)
            return c2
        lax.fori_loop(0, SC_SUP, schunk, 0)
        plsc.subcore_barrier()
        pltpu.sync_copy(ychunk.at[pl.ds(tid * DR, DR), :],
                        s_out.at[pl.ds(lo + tid * DR, DR), :])
        plsc.subcore_barrier()
        return carry
    lax.fori_loop(0, NCH // 2, one_pass, 0)


def _prescale_body(x_ref, d_ref, o_ref):
    bn = x_ref.shape[0]
    deg = d_ref[:, 0:1] + 1.0
    dis = lax.rsqrt(deg)
    o_ref[...] = jnp.concatenate(
        [dis * x_ref[...], jnp.zeros((bn, 32), jnp.float32)], axis=1)


def _finale_body(s_ref, x_ref, d_ref, az_ref, ah_ref, bz_ref, bh_ref,
                 pr_ref, lw_ref, lb_ref, o_ref):
    bnd = x_ref.shape[0]
    deg = d_ref[:, 0:1] + 1.0
    dis = lax.rsqrt(deg)
    Az = az_ref[...]
    Ah = ah_ref[...]
    acc = jnp.zeros((bnd, F_OUT), jnp.float32)
    for t in range(P):
        St = s_ref[:, t * 8:t * 8 + 8]
        Xt = x_ref[:, t * 8:t * 8 + 8]
        Yt = dis * (St + dis * Xt)
        Z = jax.nn.sigmoid(
            jnp.dot(Yt, Az, preferred_element_type=jnp.float32) + bz_ref[...])
        Ht = jnp.tanh(
            jnp.dot(Yt, Ah, preferred_element_type=jnp.float32) + bh_ref[...])
        acc = acc + pr_ref[0, t] * (1.0 - Z) * Ht
    h = jnp.maximum(acc, 0.0)
    o_ref[...] = (jnp.dot(h, lw_ref[...], preferred_element_type=jnp.float32)
                  + lb_ref[...])


def kernel(x, edge_index, edge_weight, Wz, bz, Wr, br, Wh, bh,
           LzW, Lzb, LrW, Lrb, LhW, Lhb, att, linW, linb):
    f32 = jnp.float32
    row = edge_index[0]
    col = edge_index[1]
    zpad_i = jnp.zeros((EP - E,), jnp.int32)
    row2 = jnp.concatenate([row, zpad_i]).reshape(ER, 128)
    col2 = jnp.concatenate([col, zpad_i]).reshape(ER, 128)
    w2 = jnp.concatenate([edge_weight, jnp.zeros((EP - E,), f32)]
                         ).reshape(ER, 128)
    x2t = x.transpose(0, 2, 1).reshape(N, F_IN * P)
    x2t_p = jnp.concatenate([x2t, jnp.zeros((NP - N, F_IN * P), f32)], axis=0)

    degS = _deg_kernel(col2, w2)

    BN = 1024
    xs128 = pl.pallas_call(
        _prescale_body,
        grid=(NP // BN,),
        in_specs=[pl.BlockSpec((BN, F_IN * P), lambda i: (i, 0)),
                  pl.BlockSpec((BN, 128), lambda i: (i, 0))],
        out_specs=pl.BlockSpec((BN, 128), lambda i: (i, 0)),
        out_shape=jax.ShapeDtypeStruct((NP, 128), f32),
    )(x2t_p, degS)

    s128 = _scatter_kernel(row2, col2, w2, xs128)

    Az = Wz @ LzW[:F_OUT]
    bz2 = (bz @ LzW[:F_OUT] + Lzb).reshape(1, F_OUT)
    Ah = Wh @ LhW[:F_OUT]
    bh2 = (bh @ LhW[:F_OUT] + Lhb).reshape(1, F_OUT)
    probs = jax.nn.softmax(att).reshape(1, P)
    linb2 = linb.reshape(1, P)

    BND = 2048
    out = pl.pallas_call(
        _finale_body,
        grid=(NP // BND,),
        in_specs=[pl.BlockSpec((BND, 128), lambda i: (i, 0)),
                  pl.BlockSpec((BND, F_IN * P), lambda i: (i, 0)),
                  pl.BlockSpec((BND, 128), lambda i: (i, 0)),
                  pl.BlockSpec((F_IN, F_OUT), lambda i: (0, 0)),
                  pl.BlockSpec((F_IN, F_OUT), lambda i: (0, 0)),
                  pl.BlockSpec((1, F_OUT), lambda i: (0, 0)),
                  pl.BlockSpec((1, F_OUT), lambda i: (0, 0)),
                  pl.BlockSpec((1, P), lambda i: (0, 0)),
                  pl.BlockSpec((F_OUT, P), lambda i: (0, 0)),
                  pl.BlockSpec((1, P), lambda i: (0, 0))],
        out_specs=pl.BlockSpec((BND, P), lambda i: (i, 0)),
        out_shape=jax.ShapeDtypeStruct((NP, P), f32),
    )(s128, x2t_p, degS, Az, Ah, bz2, bh2, probs, linW, linb2)
    return out[:N]


# speedup vs baseline: 12.5034x; 1.3791x over previous
import functools
import jax
import jax.numpy as jnp
from jax import lax
from jax.experimental import pallas as pl
from jax.experimental.pallas import tpu as pltpu
from jax.experimental.pallas import tpu_sc as plsc

N = 100000
E = 1600000
F_IN = 8
F_OUT = 24
P = 12
NP = 102400
EP = 1638400
ER = 12800
NCH = 10
CH = NP // NCH
CHT = CH + 128
TRASH = CH
YTR = CHT // 16
ZR = 8
DR = CH // 16
SROWS = ER // 16
SC_SUP = 50
STG = 2208

_mesh = plsc.VectorSubcoreMesh(core_axis_name="c", subcore_axis_name="s")


def _splat(vec, lane):
    idx = jnp.full((16, 1), lane, jnp.int32)
    return lax.gather(
        vec, idx,
        lax.GatherDimensionNumbers(offset_dims=(), collapsed_slice_dims=(0,),
                                   start_index_map=(0,)),
        (1,), mode=lax.GatherScatterMode.PROMISE_IN_BOUNDS)


@functools.partial(
    pl.kernel, mesh=_mesh,
    compiler_params=pltpu.CompilerParams(needs_layout_passes=False),
    out_type=jax.ShapeDtypeStruct((32 * NP,), jnp.float32),
    scratch_types=[
        pltpu.VMEM((NP,), jnp.float32),
        pltpu.VMEM((8, 128), jnp.int32),
        pltpu.VMEM((8, 128), jnp.float32),
    ],
)
def _deg_kernel(col_h, w_h, deg_out, degtile, cbuf, wbuf):
    core = lax.axis_index("c")
    tid = lax.axis_index("s")
    wid = tid * 2 + core

    def zb(i, carry):
        degtile[pl.ds(i * 16, 16)] = jnp.zeros((16,), jnp.float32)
        return carry
    lax.fori_loop(0, NP // 16, zb, 0)

    def chunk(k, carry):
        base = wid * 400 + k * 8
        pltpu.sync_copy(col_h.at[pl.ds(base, 8)], cbuf)
        pltpu.sync_copy(w_h.at[pl.ds(base, 8)], wbuf)
        for r in range(8):
            for l in range(8):
                colv = cbuf[r, pl.ds(l * 16, 16)]
                wv = wbuf[r, pl.ds(l * 16, 16)]
                plsc.addupdate_scatter(degtile, [colv], wv)
        return carry
    lax.fori_loop(0, 50, chunk, 0)
    pltpu.sync_copy(degtile, deg_out.at[pl.ds(wid * NP, NP)])


@functools.partial(
    pl.kernel, mesh=_mesh,
    compiler_params=pltpu.CompilerParams(needs_layout_passes=False),
    out_type=jax.ShapeDtypeStruct((NP, 128), jnp.float32),
    scratch_types=[
        pltpu.VMEM_SHARED((CHT, 128), jnp.float32),
        pltpu.VMEM((16, 128), jnp.int32),
        pltpu.VMEM((16, 128), jnp.int32),
        pltpu.VMEM((16, 128), jnp.float32),
        pltpu.VMEM((STG,), jnp.int32),
        pltpu.VMEM((STG,), jnp.int32),
        pltpu.VMEM((STG,), jnp.float32),
        pltpu.VMEM((128,), jnp.int32),
        pltpu.VMEM((128,), jnp.int32),
        pltpu.VMEM((128,), jnp.int32),
        pltpu.VMEM((128,), jnp.int32),
        pltpu.VMEM((128, 128), jnp.float32),
        pltpu.VMEM((128, 128), jnp.float32),
        pltpu.VMEM((ZR, 128), jnp.float32),
        pltpu.SemaphoreType.DMA,
        pltpu.SemaphoreType.DMA,
        pltpu.SemaphoreType.DMA,
        pltpu.SemaphoreType.DMA,
    ],
)
def _scatter_kernel(row_h, col_h, w_h, xs_h, s_out, ychunk,
                    rbuf, cbuf, wbuf, s_r, s_l, s_w, grow0, grow1,
                    lrow0, lrow1, rows_v0, rows_v1, zbuf,
                    semg0, semg1, sems0, sems1):
    core = lax.axis_index("c")
    tid = lax.axis_index("s")

    def zb(j, carry):
        zbuf[j // 8, pl.ds((j % 8) * 16, 16)] = jnp.zeros((16,), jnp.float32)
        return carry
    lax.fori_loop(0, ZR * 8, zb, 0)

    def one_pass(p, carry):
        fc = core * (NCH // 2) + p
        lo = fc * CH

        def zy(m, c2):
            pltpu.sync_copy(zbuf, ychunk.at[pl.ds(tid * YTR + m * ZR, ZR), :])
            return c2
        lax.fori_loop(0, YTR // ZR, zy, 0)
        plsc.subcore_barrier()

        def schunk(k, c2):
            base = tid * SROWS + k * 16
            pltpu.sync_copy(row_h.at[pl.ds(base, 16)], rbuf)
            pltpu.sync_copy(col_h.at[pl.ds(base, 16)], cbuf)
            pltpu.sync_copy(w_h.at[pl.ds(base, 16)], wbuf)
            for r in range(16):
                for l in range(8):
                    e16 = (r * 8 + l) * 16
                    colv = cbuf[r, pl.ds(l * 16, 16)]
                    rowv = rbuf[r, pl.ds(l * 16, 16)]
                    wv = wbuf[r, pl.ds(l * 16, 16)]
                    m = (colv >= lo) & (colv < lo + CH)
                    s_l[pl.ds(e16, 16)] = jnp.where(m, colv - lo, TRASH)
                    s_r[pl.ds(e16, 16)] = rowv
                    s_w[pl.ds(e16, 16)] = jnp.where(m, wv, 0.0)

            grows = (grow0, grow1)
            lrows = (lrow0, lrow1)
            rbufs = (rows_v0, rows_v1)
            semgs = (semg0, semg1)
            semss = (sems0, sems1)

            def pair(k2, c3):
                for pb in range(2):
                    b = k2 * 2 + pb
                    for i in range(8):
                        grows[pb][pl.ds(i * 16, 16)] = (
                            s_r[pl.ds(b * 128 + i * 16, 16)])
                        lrows[pb][pl.ds(i * 16, 16)] = (
                            s_l[pl.ds(b * 128 + i * 16, 16)])
                gh = [pltpu.async_copy(xs_h.at[grows[pb]], rbufs[pb],
                                       semgs[pb]) for pb in range(2)]
                sh = []
                for pb in range(2):
                    b = k2 * 2 + pb
                    cur = rbufs[pb]
                    gh[pb].wait()
                    for r in range(8):
                        wv = s_w[pl.ds(b * 128 + r * 16, 16)]
                        for l2 in range(16):
                            e = r * 16 + l2
                            s = _splat(wv, l2)
                            for q in range(6):
                                cur[e, pl.ds(q * 16, 16)] = (
                                    cur[e, pl.ds(q * 16, 16)] * s)
                    sh.append(pltpu.async_copy(cur, ychunk.at[lrows[pb]],
                                               semss[pb], add=True))
                sh[0].wait()
                sh[1].wait()
                return c3
            lax.fori_loop(0, 8, pair, 0)
            return c2
        lax.fori_loop(0, SC_SUP, schunk, 0)
        plsc.subcore_barrier()
        pltpu.sync_copy(ychunk.at[pl.ds(tid * DR, DR), :],
                        s_out.at[pl.ds(lo + tid * DR, DR), :])
        plsc.subcore_barrier()
        return carry
    lax.fori_loop(0, NCH // 2, one_pass, 0)


def _prescale_body(x_ref, d_ref, o_ref):
    bn = x_ref.shape[0]
    deg = jnp.sum(d_ref[...], axis=1, keepdims=True) + 1.0
    dis = lax.rsqrt(deg)
    o_ref[...] = jnp.concatenate(
        [dis * x_ref[...], jnp.zeros((bn, 32), jnp.float32)], axis=1)


def _finale_body(s_ref, x_ref, d_ref, az_ref, ah_ref, bz_ref, bh_ref,
                 pr_ref, lw_ref, lb_ref, o_ref):
    bnd = x_ref.shape[0]
    deg = jnp.sum(d_ref[...], axis=1, keepdims=True) + 1.0
    dis = lax.rsqrt(deg)
    Az = az_ref[...]
    Ah = ah_ref[...]
    acc = jnp.zeros((bnd, F_OUT), jnp.float32)
    for t in range(P):
        St = s_ref[:, t * 8:t * 8 + 8]
        Xt = x_ref[:, t * 8:t * 8 + 8]
        Yt = dis * (St + dis * Xt)
        Z = jax.nn.sigmoid(
            jnp.dot(Yt, Az, preferred_element_type=jnp.float32) + bz_ref[...])
        Ht = jnp.tanh(
            jnp.dot(Yt, Ah, preferred_element_type=jnp.float32) + bh_ref[...])
        acc = acc + pr_ref[0, t] * (1.0 - Z) * Ht
    h = jnp.maximum(acc, 0.0)
    o_ref[...] = (jnp.dot(h, lw_ref[...], preferred_element_type=jnp.float32)
                  + lb_ref[...])


def kernel(x, edge_index, edge_weight, Wz, bz, Wr, br, Wh, bh,
           LzW, Lzb, LrW, Lrb, LhW, Lhb, att, linW, linb):
    f32 = jnp.float32
    row = edge_index[0]
    col = edge_index[1]
    zpad_i = jnp.zeros((EP - E,), jnp.int32)
    row2 = jnp.concatenate([row, zpad_i]).reshape(ER, 128)
    col2 = jnp.concatenate([col, zpad_i]).reshape(ER, 128)
    w2 = jnp.concatenate([edge_weight, jnp.zeros((EP - E,), f32)]
                         ).reshape(ER, 128)
    x2t = x.transpose(0, 2, 1).reshape(N, F_IN * P)
    x2t_p = jnp.concatenate([x2t, jnp.zeros((NP - N, F_IN * P), f32)], axis=0)

    deg = _deg_kernel(col2, w2)
    degS = deg.reshape(32, NP).T

    BN = 1024
    xs128 = pl.pallas_call(
        _prescale_body,
        grid=(NP // BN,),
        in_specs=[pl.BlockSpec((BN, F_IN * P), lambda i: (i, 0)),
                  pl.BlockSpec((BN, 32), lambda i: (i, 0))],
        out_specs=pl.BlockSpec((BN, 128), lambda i: (i, 0)),
        out_shape=jax.ShapeDtypeStruct((NP, 128), f32),
    )(x2t_p, degS)

    s128 = _scatter_kernel(row2, col2, w2, xs128)

    Az = Wz @ LzW[:F_OUT]
    bz2 = (bz @ LzW[:F_OUT] + Lzb).reshape(1, F_OUT)
    Ah = Wh @ LhW[:F_OUT]
    bh2 = (bh @ LhW[:F_OUT] + Lhb).reshape(1, F_OUT)
    probs = jax.nn.softmax(att).reshape(1, P)
    linb2 = linb.reshape(1, P)

    BND = 2048
    out = pl.pallas_call(
        _finale_body,
        grid=(NP // BND,),
        in_specs=[pl.BlockSpec((BND, 128), lambda i: (i, 0)),
                  pl.BlockSpec((BND, F_IN * P), lambda i: (i, 0)),
                  pl.BlockSpec((BND, 32), lambda i: (i, 0)),
                  pl.BlockSpec((F_IN, F_OUT), lambda i: (0, 0)),
                  pl.BlockSpec((F_IN, F_OUT), lambda i: (0, 0)),
                  pl.BlockSpec((1, F_OUT), lambda i: (0, 0)),
                  pl.BlockSpec((1, F_OUT), lambda i: (0, 0)),
                  pl.BlockSpec((1, P), lambda i: (0, 0)),
                  pl.BlockSpec((F_OUT, P), lambda i: (0, 0)),
                  pl.BlockSpec((1, P), lambda i: (0, 0))],
        out_specs=pl.BlockSpec((BND, P), lambda i: (i, 0)),
        out_shape=jax.ShapeDtypeStruct((NP, P), f32),
    )(s128, x2t_p, degS, Az, Ah, bz2, bh2, probs, linW, linb2)
    return out[:N]


# speedup vs baseline: 15.7781x; 1.2619x over previous
import functools
import jax
import jax.numpy as jnp
from jax import lax
from jax.experimental import pallas as pl
from jax.experimental.pallas import tpu as pltpu
from jax.experimental.pallas import tpu_sc as plsc

N = 100000
E = 1600000
F_IN = 8
F_OUT = 24
P = 12
NP = 102400
EP = 1638400
ER = 12800
NCH = 10
CH = NP // NCH
CHT = CH + 128
TRASH = CH
YTR = CHT // 16
ZR = 8
DR = CH // 16
SROWS = ER // 16
SC_SUP = 50
STG = 2208

_mesh = plsc.VectorSubcoreMesh(core_axis_name="c", subcore_axis_name="s")


def _splat(vec, lane):
    idx = jnp.full((16, 1), lane, jnp.int32)
    return lax.gather(
        vec, idx,
        lax.GatherDimensionNumbers(offset_dims=(), collapsed_slice_dims=(0,),
                                   start_index_map=(0,)),
        (1,), mode=lax.GatherScatterMode.PROMISE_IN_BOUNDS)


@functools.partial(
    pl.kernel, mesh=_mesh,
    compiler_params=pltpu.CompilerParams(needs_layout_passes=False),
    out_type=jax.ShapeDtypeStruct((32 * NP,), jnp.float32),
    scratch_types=[
        pltpu.VMEM((NP,), jnp.float32),
        pltpu.VMEM((8, 128), jnp.int32),
        pltpu.VMEM((8, 128), jnp.float32),
    ],
)
def _deg_kernel(col_h, w_h, deg_out, degtile, cbuf, wbuf):
    core = lax.axis_index("c")
    tid = lax.axis_index("s")
    wid = tid * 2 + core

    def zb(i, carry):
        degtile[pl.ds(i * 16, 16)] = jnp.zeros((16,), jnp.float32)
        return carry
    lax.fori_loop(0, NP // 16, zb, 0)

    def chunk(k, carry):
        base = wid * 400 + k * 8
        pltpu.sync_copy(col_h.at[pl.ds(base, 8)], cbuf)
        pltpu.sync_copy(w_h.at[pl.ds(base, 8)], wbuf)
        for r in range(8):
            for l in range(8):
                colv = cbuf[r, pl.ds(l * 16, 16)]
                wv = wbuf[r, pl.ds(l * 16, 16)]
                plsc.addupdate_scatter(degtile, [colv], wv)
        return carry
    lax.fori_loop(0, 50, chunk, 0)
    pltpu.sync_copy(degtile, deg_out.at[pl.ds(wid * NP, NP)])


@functools.partial(
    pl.kernel, mesh=_mesh,
    compiler_params=pltpu.CompilerParams(needs_layout_passes=False),
    out_type=jax.ShapeDtypeStruct((NP, 128), jnp.float32),
    scratch_types=[
        pltpu.VMEM_SHARED((CHT, 128), jnp.float32),
        pltpu.VMEM((16, 128), jnp.int32),
        pltpu.VMEM((16, 128), jnp.int32),
        pltpu.VMEM((16, 128), jnp.float32),
        pltpu.VMEM((STG,), jnp.int32),
        pltpu.VMEM((STG,), jnp.int32),
        pltpu.VMEM((STG,), jnp.float32),
        pltpu.VMEM((128,), jnp.int32),
        pltpu.VMEM((128,), jnp.int32),
        pltpu.VMEM((128,), jnp.int32),
        pltpu.VMEM((128,), jnp.int32),
        pltpu.VMEM((128, 128), jnp.float32),
        pltpu.VMEM((128, 128), jnp.float32),
        pltpu.VMEM((ZR, 128), jnp.float32),
        pltpu.SemaphoreType.DMA,
        pltpu.SemaphoreType.DMA,
        pltpu.SemaphoreType.DMA,
        pltpu.SemaphoreType.DMA,
    ],
)
def _scatter_kernel(row_h, col_h, w_h, xs_h, s_out, ychunk,
                    rbuf, cbuf, wbuf, s_r, s_l, s_w, grow0, grow1,
                    lrow0, lrow1, rows_v0, rows_v1, zbuf,
                    semg0, semg1, sems0, sems1):
    core = lax.axis_index("c")
    tid = lax.axis_index("s")

    def zb(j, carry):
        zbuf[j // 8, pl.ds((j % 8) * 16, 16)] = jnp.zeros((16,), jnp.float32)
        return carry
    lax.fori_loop(0, ZR * 8, zb, 0)

    def one_pass(p, carry):
        fc = core * (NCH // 2) + p
        lo = fc * CH

        def zy(m, c2):
            pltpu.sync_copy(zbuf, ychunk.at[pl.ds(tid * YTR + m * ZR, ZR), :])
            return c2
        lax.fori_loop(0, YTR // ZR, zy, 0)
        plsc.subcore_barrier()

        def schunk(k, c2):
            base = tid * SROWS + k * 16
            pltpu.sync_copy(row_h.at[pl.ds(base, 16)], rbuf)
            pltpu.sync_copy(col_h.at[pl.ds(base, 16)], cbuf)
            pltpu.sync_copy(w_h.at[pl.ds(base, 16)], wbuf)
            cnt = jnp.int32(0)
            iota16 = lax.iota(jnp.int32, 16)
            for r in range(16):
                for l in range(8):
                    colv = cbuf[r, pl.ds(l * 16, 16)]
                    rowv = rbuf[r, pl.ds(l * 16, 16)]
                    wv = wbuf[r, pl.ds(l * 16, 16)]
                    m = (colv >= lo) & (colv < lo + CH)
                    cs = plsc.cumsum(m.astype(jnp.int32))
                    pos = cs + (cnt - 1)
                    plsc.store_scatter(s_l, [pos], colv - lo, mask=m)
                    plsc.store_scatter(s_r, [pos], rowv, mask=m)
                    plsc.store_scatter(s_w, [pos], wv, mask=m)
                    cnt = cnt + jnp.max(cs)
            for kk in range(8):
                pos = iota16 + (cnt + kk * 16)
                plsc.store_scatter(s_l, [pos], jnp.full((16,), TRASH,
                                                        jnp.int32))
                plsc.store_scatter(s_r, [pos], jnp.zeros((16,), jnp.int32))
                plsc.store_scatter(s_w, [pos], jnp.zeros((16,), jnp.float32))

            grows = (grow0, grow1)
            lrows = (lrow0, lrow1)
            rbufs = (rows_v0, rows_v1)
            semgs = (semg0, semg1)
            semss = (sems0, sems1)

            def pair(k2, c3):
                for pb in range(2):
                    b = k2 * 2 + pb

                    @pl.when(b * 128 < cnt)
                    def _():
                        cur = rbufs[pb]
                        for i in range(8):
                            grows[pb][pl.ds(i * 16, 16)] = (
                                s_r[pl.ds(b * 128 + i * 16, 16)])
                            lrows[pb][pl.ds(i * 16, 16)] = (
                                s_l[pl.ds(b * 128 + i * 16, 16)])
                        pltpu.async_copy(xs_h.at[grows[pb]], cur,
                                         semgs[pb]).wait()
                        for r in range(8):
                            wv = s_w[pl.ds(b * 128 + r * 16, 16)]
                            for l2 in range(16):
                                e = r * 16 + l2
                                s = _splat(wv, l2)
                                for q in range(6):
                                    cur[e, pl.ds(q * 16, 16)] = (
                                        cur[e, pl.ds(q * 16, 16)] * s)
                        pltpu.sync_copy(cur, ychunk.at[lrows[pb]], add=True)
                return c3
            lax.fori_loop(0, 8, pair, 0)
            return c2
        lax.fori_loop(0, SC_SUP, schunk, 0)
        plsc.subcore_barrier()
        pltpu.sync_copy(ychunk.at[pl.ds(tid * DR, DR), :],
                        s_out.at[pl.ds(lo + tid * DR, DR), :])
        plsc.subcore_barrier()
        return carry
    lax.fori_loop(0, NCH // 2, one_pass, 0)


def _prescale_body(x_ref, d_ref, o_ref):
    bn = x_ref.shape[0]
    deg = jnp.sum(d_ref[...], axis=1, keepdims=True) + 1.0
    dis = lax.rsqrt(deg)
    o_ref[...] = jnp.concatenate(
        [dis * x_ref[...], jnp.zeros((bn, 32), jnp.float32)], axis=1)


def _finale_body(s_ref, x_ref, d_ref, az_ref, ah_ref, bz_ref, bh_ref,
                 pr_ref, lw_ref, lb_ref, o_ref):
    bnd = x_ref.shape[0]
    deg = jnp.sum(d_ref[...], axis=1, keepdims=True) + 1.0
    dis = lax.rsqrt(deg)
    Az = az_ref[...]
    Ah = ah_ref[...]
    acc = jnp.zeros((bnd, F_OUT), jnp.float32)
    for t in range(P):
        St = s_ref[:, t * 8:t * 8 + 8]
        Xt = x_ref[:, t * 8:t * 8 + 8]
        Yt = dis * (St + dis * Xt)
        Z = jax.nn.sigmoid(
            jnp.dot(Yt, Az, preferred_element_type=jnp.float32) + bz_ref[...])
        Ht = jnp.tanh(
            jnp.dot(Yt, Ah, preferred_element_type=jnp.float32) + bh_ref[...])
        acc = acc + pr_ref[0, t] * (1.0 - Z) * Ht
    h = jnp.maximum(acc, 0.0)
    o_ref[...] = (jnp.dot(h, lw_ref[...], preferred_element_type=jnp.float32)
                  + lb_ref[...])


def kernel(x, edge_index, edge_weight, Wz, bz, Wr, br, Wh, bh,
           LzW, Lzb, LrW, Lrb, LhW, Lhb, att, linW, linb):
    f32 = jnp.float32
    row = edge_index[0]
    col = edge_index[1]
    zpad_i = jnp.zeros((EP - E,), jnp.int32)
    row2 = jnp.concatenate([row, zpad_i]).reshape(ER, 128)
    col2 = jnp.concatenate([col, zpad_i]).reshape(ER, 128)
    w2 = jnp.concatenate([edge_weight, jnp.zeros((EP - E,), f32)]
                         ).reshape(ER, 128)
    x2t = x.transpose(0, 2, 1).reshape(N, F_IN * P)
    x2t_p = jnp.concatenate([x2t, jnp.zeros((NP - N, F_IN * P), f32)], axis=0)

    deg = _deg_kernel(col2, w2)
    degS = deg.reshape(32, NP).T

    BN = 1024
    xs128 = pl.pallas_call(
        _prescale_body,
        grid=(NP // BN,),
        in_specs=[pl.BlockSpec((BN, F_IN * P), lambda i: (i, 0)),
                  pl.BlockSpec((BN, 32), lambda i: (i, 0))],
        out_specs=pl.BlockSpec((BN, 128), lambda i: (i, 0)),
        out_shape=jax.ShapeDtypeStruct((NP, 128), f32),
    )(x2t_p, degS)

    s128 = _scatter_kernel(row2, col2, w2, xs128)

    Az = Wz @ LzW[:F_OUT]
    bz2 = (bz @ LzW[:F_OUT] + Lzb).reshape(1, F_OUT)
    Ah = Wh @ LhW[:F_OUT]
    bh2 = (bh @ LhW[:F_OUT] + Lhb).reshape(1, F_OUT)
    probs = jax.nn.softmax(att).reshape(1, P)
    linb2 = linb.reshape(1, P)

    BND = 2048
    out = pl.pallas_call(
        _finale_body,
        grid=(NP // BND,),
        in_specs=[pl.BlockSpec((BND, 128), lambda i: (i, 0)),
                  pl.BlockSpec((BND, F_IN * P), lambda i: (i, 0)),
                  pl.BlockSpec((BND, 32), lambda i: (i, 0)),
                  pl.BlockSpec((F_IN, F_OUT), lambda i: (0, 0)),
                  pl.BlockSpec((F_IN, F_OUT), lambda i: (0, 0)),
                  pl.BlockSpec((1, F_OUT), lambda i: (0, 0)),
                  pl.BlockSpec((1, F_OUT), lambda i: (0, 0)),
                  pl.BlockSpec((1, P), lambda i: (0, 0)),
                  pl.BlockSpec((F_OUT, P), lambda i: (0, 0)),
                  pl.BlockSpec((1, P), lambda i: (0, 0))],
        out_specs=pl.BlockSpec((BND, P), lambda i: (i, 0)),
        out_shape=jax.ShapeDtypeStruct((NP, P), f32),
    )(s128, x2t_p, degS, Az, Ah, bz2, bh2, probs, linW, linb2)
    return out[:N]
